# baseline (device time: 177792 ns/iter reference)
import jax
import jax.numpy as jnp
from jax import lax
from jax.experimental import pallas as pl
from jax.experimental.pallas import tpu as pltpu

N_DEV = 8
SEQ = 2048
CHUNK = 256
D = 1024
NH = 8
DH = 128
SCALE = 0.08838834764831843

F32 = jnp.float32
BF16 = jnp.bfloat16


def kernel(x, Wq, Wo, Wk, Wv):
    x2 = x.reshape(CHUNK, D)

    def body(x_ref, wq_ref, wo_ref, wk_ref, wv_ref, out_ref,
             xg_ref, qf_ref, kf_ref, vf_ref, oblk_ref, part_ref, sbuf_ref,
             ag_send, ag_recv, rs_send, rs_recv, mid_sem):
        my = lax.axis_index("i")
        right = lax.rem(my + 1, N_DEV)
        left = lax.rem(my + N_DEV - 1, N_DEV)

        barrier = pltpu.get_barrier_semaphore()
        for nbr in (left, right):
            pl.semaphore_signal(barrier, inc=1, device_id=(nbr,),
                                device_id_type=pl.DeviceIdType.MESH)
        pl.semaphore_wait(barrier, 2)

        def qkv_chunk(c):
            row = c * CHUNK
            xb = xg_ref[pl.ds(row, CHUNK), :].astype(F32)
            qf_ref[pl.ds(row, CHUNK), :] = jnp.dot(
                xb, wq_ref[...], preferred_element_type=F32).astype(BF16)
            kf_ref[pl.ds(row, CHUNK), :] = jnp.dot(
                xb, wk_ref[...], preferred_element_type=F32).astype(BF16)
            vf_ref[pl.ds(row, CHUNK), :] = jnp.dot(
                xb, wv_ref[...], preferred_element_type=F32)

        xg_ref[pl.ds(my * CHUNK, CHUNK), :] = x_ref[...].astype(BF16)
        ag = []
        d0 = pltpu.make_async_remote_copy(
            src_ref=xg_ref.at[pl.ds(my * CHUNK, CHUNK), :],
            dst_ref=xg_ref.at[pl.ds(my * CHUNK, CHUNK), :],
            send_sem=ag_send.at[0], recv_sem=ag_recv.at[0],
            device_id=(right,), device_id_type=pl.DeviceIdType.MESH,
        )
        d0.start()
        ag.append(d0)
        qkv_chunk(my)
        for h in range(N_DEV - 1):
            ag[h].wait_recv()
            c = lax.rem(my - 1 - h + N_DEV, N_DEV)
            if h < N_DEV - 2:
                d = pltpu.make_async_remote_copy(
                    src_ref=xg_ref.at[pl.ds(c * CHUNK, CHUNK), :],
                    dst_ref=xg_ref.at[pl.ds(c * CHUNK, CHUNK), :],
                    send_sem=ag_send.at[h + 1], recv_sem=ag_recv.at[h + 1],
                    device_id=(right,), device_id_type=pl.DeviceIdType.MESH,
                )
                d.start()
                ag.append(d)
            qkv_chunk(c)
        for d in ag:
            d.wait_send()

        pl.semaphore_signal(mid_sem, inc=1, device_id=(left,),
                            device_id_type=pl.DeviceIdType.MESH)

        def attn_chunk(c):
            row = c * CHUNK

            def hb(hh, carry):
                hcol = hh * DH
                q = qf_ref[pl.ds(row, CHUNK), pl.ds(hcol, DH)]
                s = lax.dot_general(
                    q, kf_ref[:, pl.ds(hcol, DH)], (((1,), (1,)), ((), ())),
                    preferred_element_type=F32) * SCALE
                m = jnp.max(s, axis=1, keepdims=True)
                p = jnp.exp(s - m)
                l = jnp.sum(p, axis=1, keepdims=True)
                ob = jnp.dot(p, vf_ref[:, pl.ds(hcol, DH)],
                             preferred_element_type=F32) / l
                oblk_ref[:, pl.ds(hcol, DH)] = ob
                return carry

            lax.fori_loop(0, NH, hb, 0)
            part_ref[pl.ds(row, CHUNK), :] = jnp.dot(
                oblk_ref[...], wo_ref[...], preferred_element_type=F32)

        rs = []
        for st in range(N_DEV - 1):
            c = lax.rem(my - 1 - st + N_DEV, N_DEV)
            attn_chunk(c)
            if st == 0:
                sbuf_ref[0] = part_ref[pl.ds(c * CHUNK, CHUNK), :].astype(BF16)
                pl.semaphore_wait(mid_sem, 1)
            else:
                rs[st - 1].wait_recv()
                sbuf_ref[st] = (
                    part_ref[pl.ds(c * CHUNK, CHUNK), :]
                    + xg_ref[(st - 1) * CHUNK:st * CHUNK, :].astype(F32)
                ).astype(BF16)
            r = pltpu.make_async_remote_copy(
                src_ref=sbuf_ref.at[st],
                dst_ref=xg_ref.at[st * CHUNK:(st + 1) * CHUNK, :],
                send_sem=rs_send.at[st], recv_sem=rs_recv.at[st],
                device_id=(right,), device_id_type=pl.DeviceIdType.MESH,
            )
            r.start()
            rs.append(r)
        attn_chunk(my)
        rs[N_DEV - 2].wait_recv()
        out_ref[...] = (
            part_ref[pl.ds(my * CHUNK, CHUNK), :]
            + xg_ref[(N_DEV - 2) * CHUNK:(N_DEV - 1) * CHUNK, :].astype(F32))
        for r in rs:
            r.wait_send()

    out = pl.pallas_call(
        body,
        out_shape=jax.ShapeDtypeStruct((CHUNK, D), jnp.float32),
        in_specs=[pl.BlockSpec(memory_space=pltpu.VMEM)] * 5,
        out_specs=pl.BlockSpec(memory_space=pltpu.VMEM),
        scratch_shapes=[
            pltpu.VMEM((SEQ, D), BF16),
            pltpu.VMEM((SEQ, D), BF16),
            pltpu.VMEM((SEQ, D), BF16),
            pltpu.VMEM((SEQ, D), F32),
            pltpu.VMEM((CHUNK, D), F32),
            pltpu.VMEM((SEQ, D), F32),
            pltpu.VMEM((N_DEV - 1, CHUNK, D), BF16),
            pltpu.SemaphoreType.DMA((N_DEV - 1,)),
            pltpu.SemaphoreType.DMA((N_DEV - 1,)),
            pltpu.SemaphoreType.DMA((N_DEV - 1,)),
            pltpu.SemaphoreType.DMA((N_DEV - 1,)),
            pltpu.SemaphoreType.REGULAR,
        ],
        compiler_params=pltpu.CompilerParams(
            collective_id=0, vmem_limit_bytes=100 * 1024 * 1024),
    )(x2, Wq, Wo, Wk, Wv)
    return out.reshape(1, CHUNK, D)


# device time: 130325 ns/iter; 1.3642x vs baseline; 1.3642x over previous
import jax
import jax.numpy as jnp
from jax import lax
from jax.experimental import pallas as pl
from jax.experimental.pallas import tpu as pltpu

N_DEV = 8
SEQ = 2048
CHUNK = 256
D = 1024
NH = 8
DH = 128
SCALE = 0.08838834764831843
QSCALE = SCALE * 1.4426950408889634

F32 = jnp.float32
BF16 = jnp.bfloat16


def kernel(x, Wq, Wo, Wk, Wv):
    x2 = x.reshape(CHUNK, D)

    def body(x_ref, wq_ref, wo_ref, wk_ref, wv_ref, out_ref,
             xg_ref, qf_ref, kf_ref, vf_ref, oblk_ref, part_ref, sbuf_ref,
             ag_send, ag_recv, rs_send, rs_recv, mid_sem):
        my = lax.axis_index("i")
        right = lax.rem(my + 1, N_DEV)
        left = lax.rem(my + N_DEV - 1, N_DEV)

        barrier = pltpu.get_barrier_semaphore()
        for nbr in (left, right):
            pl.semaphore_signal(barrier, inc=1, device_id=(nbr,),
                                device_id_type=pl.DeviceIdType.MESH)
        pl.semaphore_wait(barrier, 2)

        def qkv_chunk(c):
            row = c * CHUNK
            xb = xg_ref[pl.ds(row, CHUNK), :].astype(F32)
            qf_ref[pl.ds(row, CHUNK), :] = (jnp.dot(
                xb, wq_ref[...], preferred_element_type=F32)
                * QSCALE).astype(BF16)
            kf_ref[pl.ds(row, CHUNK), :] = jnp.dot(
                xb, wk_ref[...], preferred_element_type=F32).astype(BF16)
            vf_ref[pl.ds(row, CHUNK), :] = jnp.dot(
                xb, wv_ref[...], preferred_element_type=F32)

        xg_ref[pl.ds(my * CHUNK, CHUNK), :] = x_ref[...].astype(BF16)
        ag = []
        d0 = pltpu.make_async_remote_copy(
            src_ref=xg_ref.at[pl.ds(my * CHUNK, CHUNK), :],
            dst_ref=xg_ref.at[pl.ds(my * CHUNK, CHUNK), :],
            send_sem=ag_send.at[0], recv_sem=ag_recv.at[0],
            device_id=(right,), device_id_type=pl.DeviceIdType.MESH,
        )
        d0.start()
        ag.append(d0)
        qkv_chunk(my)
        for h in range(N_DEV - 1):
            ag[h].wait_recv()
            c = lax.rem(my - 1 - h + N_DEV, N_DEV)
            if h < N_DEV - 2:
                d = pltpu.make_async_remote_copy(
                    src_ref=xg_ref.at[pl.ds(c * CHUNK, CHUNK), :],
                    dst_ref=xg_ref.at[pl.ds(c * CHUNK, CHUNK), :],
                    send_sem=ag_send.at[h + 1], recv_sem=ag_recv.at[h + 1],
                    device_id=(right,), device_id_type=pl.DeviceIdType.MESH,
                )
                d.start()
                ag.append(d)
            qkv_chunk(c)
        for d in ag:
            d.wait_send()

        pl.semaphore_signal(mid_sem, inc=1, device_id=(left,),
                            device_id_type=pl.DeviceIdType.MESH)

        def attn_chunk(c):
            row = c * CHUNK

            def hb(hh, carry):
                hcol = hh * DH
                q = qf_ref[pl.ds(row, CHUNK), pl.ds(hcol, DH)]
                s = lax.dot_general(
                    q, kf_ref[:, pl.ds(hcol, DH)], (((1,), (1,)), ((), ())),
                    preferred_element_type=F32)
                p = jnp.exp2(s)
                l = jnp.sum(p, axis=1, keepdims=True)
                ob = jnp.dot(p, vf_ref[:, pl.ds(hcol, DH)],
                             preferred_element_type=F32)
                oblk_ref[:, pl.ds(hcol, DH)] = ob * (1.0 / l)
                return carry

            lax.fori_loop(0, NH, hb, 0)
            part_ref[pl.ds(row, CHUNK), :] = jnp.dot(
                oblk_ref[...], wo_ref[...], preferred_element_type=F32)

        rs = []
        for st in range(N_DEV - 1):
            c = lax.rem(my - 1 - st + N_DEV, N_DEV)
            attn_chunk(c)
            if st == 0:
                sbuf_ref[0] = part_ref[pl.ds(c * CHUNK, CHUNK), :].astype(BF16)
                pl.semaphore_wait(mid_sem, 1)
            else:
                rs[st - 1].wait_recv()
                sbuf_ref[st] = (
                    part_ref[pl.ds(c * CHUNK, CHUNK), :]
                    + xg_ref[(st - 1) * CHUNK:st * CHUNK, :].astype(F32)
                ).astype(BF16)
            r = pltpu.make_async_remote_copy(
                src_ref=sbuf_ref.at[st],
                dst_ref=xg_ref.at[st * CHUNK:(st + 1) * CHUNK, :],
                send_sem=rs_send.at[st], recv_sem=rs_recv.at[st],
                device_id=(right,), device_id_type=pl.DeviceIdType.MESH,
            )
            r.start()
            rs.append(r)
        attn_chunk(my)
        rs[N_DEV - 2].wait_recv()
        out_ref[...] = (
            part_ref[pl.ds(my * CHUNK, CHUNK), :]
            + xg_ref[(N_DEV - 2) * CHUNK:(N_DEV - 1) * CHUNK, :].astype(F32))
        for r in rs:
            r.wait_send()

    out = pl.pallas_call(
        body,
        out_shape=jax.ShapeDtypeStruct((CHUNK, D), jnp.float32),
        in_specs=[pl.BlockSpec(memory_space=pltpu.VMEM)] * 5,
        out_specs=pl.BlockSpec(memory_space=pltpu.VMEM),
        scratch_shapes=[
            pltpu.VMEM((SEQ, D), BF16),
            pltpu.VMEM((SEQ, D), BF16),
            pltpu.VMEM((SEQ, D), BF16),
            pltpu.VMEM((SEQ, D), F32),
            pltpu.VMEM((CHUNK, D), F32),
            pltpu.VMEM((SEQ, D), F32),
            pltpu.VMEM((N_DEV - 1, CHUNK, D), BF16),
            pltpu.SemaphoreType.DMA((N_DEV - 1,)),
            pltpu.SemaphoreType.DMA((N_DEV - 1,)),
            pltpu.SemaphoreType.DMA((N_DEV - 1,)),
            pltpu.SemaphoreType.DMA((N_DEV - 1,)),
            pltpu.SemaphoreType.REGULAR,
        ],
        compiler_params=pltpu.CompilerParams(
            collective_id=0, vmem_limit_bytes=100 * 1024 * 1024),
    )(x2, Wq, Wo, Wk, Wv)
    return out.reshape(1, CHUNK, D)


# device time: 126403 ns/iter; 1.4065x vs baseline; 1.0310x over previous
import jax
import jax.numpy as jnp
from jax import lax
from jax.experimental import pallas as pl
from jax.experimental.pallas import tpu as pltpu

N_DEV = 8
SEQ = 2048
CHUNK = 256
D = 1024
NH = 8
DH = 128
SCALE = 0.08838834764831843
QSCALE = SCALE * 1.4426950408889634

F32 = jnp.float32
BF16 = jnp.bfloat16


def kernel(x, Wq, Wo, Wk, Wv):
    x2 = x.reshape(CHUNK, D)

    def body(x_ref, wq_ref, wo_ref, wk_ref, wv_ref, out_ref,
             xg_ref, qf_ref, kf_ref, vf_ref, oblk_ref, part_ref, sbuf_ref,
             ag_send, ag_recv, rs_send, rs_recv, mid_sem):
        my = lax.axis_index("i")
        right = lax.rem(my + 1, N_DEV)
        left = lax.rem(my + N_DEV - 1, N_DEV)

        barrier = pltpu.get_barrier_semaphore()
        for nbr in (left, right):
            pl.semaphore_signal(barrier, inc=1, device_id=(nbr,),
                                device_id_type=pl.DeviceIdType.MESH)
        pl.semaphore_wait(barrier, 2)

        def qkv_chunk(c):
            row = c * CHUNK
            xb = xg_ref[pl.ds(row, CHUNK), :].astype(F32)
            qf_ref[pl.ds(row, CHUNK), :] = (jnp.dot(
                xb, wq_ref[...], preferred_element_type=F32)
                * QSCALE).astype(BF16)
            kf_ref[pl.ds(row, CHUNK), :] = jnp.dot(
                xb, wk_ref[...], preferred_element_type=F32).astype(BF16)
            vf_ref[pl.ds(row, CHUNK), :] = jnp.dot(
                xb, wv_ref[...], preferred_element_type=F32)

        xg_ref[pl.ds(my * CHUNK, CHUNK), :] = x_ref[...].astype(BF16)
        ag = []
        d0 = pltpu.make_async_remote_copy(
            src_ref=xg_ref.at[pl.ds(my * CHUNK, CHUNK), :],
            dst_ref=xg_ref.at[pl.ds(my * CHUNK, CHUNK), :],
            send_sem=ag_send.at[0], recv_sem=ag_recv.at[0],
            device_id=(right,), device_id_type=pl.DeviceIdType.MESH,
        )
        d0.start()
        ag.append(d0)
        qkv_chunk(my)
        for h in range(N_DEV - 1):
            ag[h].wait_recv()
            c = lax.rem(my - 1 - h + N_DEV, N_DEV)
            if h < N_DEV - 2:
                d = pltpu.make_async_remote_copy(
                    src_ref=xg_ref.at[pl.ds(c * CHUNK, CHUNK), :],
                    dst_ref=xg_ref.at[pl.ds(c * CHUNK, CHUNK), :],
                    send_sem=ag_send.at[h + 1], recv_sem=ag_recv.at[h + 1],
                    device_id=(right,), device_id_type=pl.DeviceIdType.MESH,
                )
                d.start()
                ag.append(d)
            qkv_chunk(c)
        for d in ag:
            d.wait_send()

        pl.semaphore_signal(mid_sem, inc=1, device_id=(left,),
                            device_id_type=pl.DeviceIdType.MESH)

        def attn_chunk(c):
            row = c * CHUNK

            for hh in range(NH):
                hcol = hh * DH
                q = qf_ref[pl.ds(row, CHUNK), pl.ds(hcol, DH)]
                s = lax.dot_general(
                    q, kf_ref[:, pl.ds(hcol, DH)], (((1,), (1,)), ((), ())),
                    preferred_element_type=F32)
                p = jnp.exp2(s)
                l = jnp.sum(p, axis=1, keepdims=True)
                ob = jnp.dot(p, vf_ref[:, pl.ds(hcol, DH)],
                             preferred_element_type=F32)
                oblk_ref[:, pl.ds(hcol, DH)] = ob * (1.0 / l)

            part_ref[pl.ds(row, CHUNK), :] = jnp.dot(
                oblk_ref[...], wo_ref[...], preferred_element_type=F32)

        rs = []
        for st in range(N_DEV - 1):
            c = lax.rem(my - 1 - st + N_DEV, N_DEV)
            attn_chunk(c)
            if st == 0:
                sbuf_ref[0] = part_ref[pl.ds(c * CHUNK, CHUNK), :].astype(BF16)
                pl.semaphore_wait(mid_sem, 1)
            else:
                rs[st - 1].wait_recv()
                sbuf_ref[st] = (
                    part_ref[pl.ds(c * CHUNK, CHUNK), :]
                    + xg_ref[(st - 1) * CHUNK:st * CHUNK, :].astype(F32)
                ).astype(BF16)
            r = pltpu.make_async_remote_copy(
                src_ref=sbuf_ref.at[st],
                dst_ref=xg_ref.at[st * CHUNK:(st + 1) * CHUNK, :],
                send_sem=rs_send.at[st], recv_sem=rs_recv.at[st],
                device_id=(right,), device_id_type=pl.DeviceIdType.MESH,
            )
            r.start()
            rs.append(r)
        attn_chunk(my)
        rs[N_DEV - 2].wait_recv()
        out_ref[...] = (
            part_ref[pl.ds(my * CHUNK, CHUNK), :]
            + xg_ref[(N_DEV - 2) * CHUNK:(N_DEV - 1) * CHUNK, :].astype(F32))
        for r in rs:
            r.wait_send()

    out = pl.pallas_call(
        body,
        out_shape=jax.ShapeDtypeStruct((CHUNK, D), jnp.float32),
        in_specs=[pl.BlockSpec(memory_space=pltpu.VMEM)] * 5,
        out_specs=pl.BlockSpec(memory_space=pltpu.VMEM),
        scratch_shapes=[
            pltpu.VMEM((SEQ, D), BF16),
            pltpu.VMEM((SEQ, D), BF16),
            pltpu.VMEM((SEQ, D), BF16),
            pltpu.VMEM((SEQ, D), F32),
            pltpu.VMEM((CHUNK, D), F32),
            pltpu.VMEM((SEQ, D), F32),
            pltpu.VMEM((N_DEV - 1, CHUNK, D), BF16),
            pltpu.SemaphoreType.DMA((N_DEV - 1,)),
            pltpu.SemaphoreType.DMA((N_DEV - 1,)),
            pltpu.SemaphoreType.DMA((N_DEV - 1,)),
            pltpu.SemaphoreType.DMA((N_DEV - 1,)),
            pltpu.SemaphoreType.REGULAR,
        ],
        compiler_params=pltpu.CompilerParams(
            collective_id=0, vmem_limit_bytes=100 * 1024 * 1024),
    )(x2, Wq, Wo, Wk, Wv)
    return out.reshape(1, CHUNK, D)


# device time: 103566 ns/iter; 1.7167x vs baseline; 1.2205x over previous
import jax
import jax.numpy as jnp
from jax import lax
from jax.experimental import pallas as pl
from jax.experimental.pallas import tpu as pltpu

N_DEV = 8
SEQ = 2048
CHUNK = 256
D = 1024
NH = 8
DH = 128
SCALE = 0.08838834764831843
QSCALE = SCALE * 1.4426950408889634

F32 = jnp.float32
BF16 = jnp.bfloat16


def kernel(x, Wq, Wo, Wk, Wv):
    x2 = x.reshape(CHUNK, D)

    def body(x_ref, wq_ref, wo_ref, wk_ref, wv_ref, out_ref,
             xg_ref, qf_ref, kf_ref, vf_ref, oblk_ref, part_ref, sbuf_ref,
             ag_send, ag_recv, al_send, al_recv, rs_send, rs_recv, mid_sem):
        my = lax.axis_index("i")
        right = lax.rem(my + 1, N_DEV)
        left = lax.rem(my + N_DEV - 1, N_DEV)

        barrier = pltpu.get_barrier_semaphore()
        for nbr in (left, right):
            pl.semaphore_signal(barrier, inc=1, device_id=(nbr,),
                                device_id_type=pl.DeviceIdType.MESH)
        pl.semaphore_wait(barrier, 2)

        def qkv_chunk(c):
            row = c * CHUNK
            xb = xg_ref[pl.ds(row, CHUNK), :].astype(F32)
            qf_ref[pl.ds(row, CHUNK), :] = (jnp.dot(
                xb, wq_ref[...], preferred_element_type=F32)
                * QSCALE).astype(BF16)
            kf_ref[pl.ds(row, CHUNK), :] = jnp.dot(
                xb, wk_ref[...], preferred_element_type=F32).astype(BF16)
            vf_ref[pl.ds(row, CHUNK), :] = jnp.dot(
                xb, wv_ref[...], preferred_element_type=F32)

        HR, HL = 4, 3

        def hop(src_chunk, sems, h, dst):
            d = pltpu.make_async_remote_copy(
                src_ref=xg_ref.at[pl.ds(src_chunk * CHUNK, CHUNK), :],
                dst_ref=xg_ref.at[pl.ds(src_chunk * CHUNK, CHUNK), :],
                send_sem=sems[0].at[h], recv_sem=sems[1].at[h],
                device_id=(dst,), device_id_type=pl.DeviceIdType.MESH,
            )
            d.start()
            return d

        xg_ref[pl.ds(my * CHUNK, CHUNK), :] = x_ref[...].astype(BF16)
        agr = [hop(my, (ag_send, ag_recv), 0, right)]
        agl = [hop(my, (al_send, al_recv), 0, left)]
        qkv_chunk(my)
        for h in range(HR):
            agr[h].wait_recv()
            cr = lax.rem(my - 1 - h + N_DEV, N_DEV)
            if h < HR - 1:
                agr.append(hop(cr, (ag_send, ag_recv), h + 1, right))
            qkv_chunk(cr)
            if h < HL:
                agl[h].wait_recv()
                cl = lax.rem(my + 1 + h, N_DEV)
                if h < HL - 1:
                    agl.append(hop(cl, (al_send, al_recv), h + 1, left))
                qkv_chunk(cl)
        for d in agr + agl:
            d.wait_send()

        pl.semaphore_signal(mid_sem, inc=1, device_id=(left,),
                            device_id_type=pl.DeviceIdType.MESH)

        def attn_chunk(c):
            row = c * CHUNK

            def hb(hh, carry):
                hcol = hh * DH
                q = qf_ref[pl.ds(row, CHUNK), pl.ds(hcol, DH)]
                s = lax.dot_general(
                    q, kf_ref[:, pl.ds(hcol, DH)], (((1,), (1,)), ((), ())),
                    preferred_element_type=F32)
                p = jnp.exp2(s)
                l = jnp.sum(p, axis=1, keepdims=True)
                ob = jnp.dot(p, vf_ref[:, pl.ds(hcol, DH)],
                             preferred_element_type=F32)
                oblk_ref[:, pl.ds(hcol, DH)] = ob * (1.0 / l)
                return carry

            lax.fori_loop(0, NH, hb, 0, unroll=2)
            part_ref[pl.ds(row, CHUNK), :] = jnp.dot(
                oblk_ref[...], wo_ref[...], preferred_element_type=F32)

        rs = []
        for st in range(N_DEV - 1):
            c = lax.rem(my - 1 - st + N_DEV, N_DEV)
            attn_chunk(c)
            if st == 0:
                sbuf_ref[0] = part_ref[pl.ds(c * CHUNK, CHUNK), :].astype(BF16)
                pl.semaphore_wait(mid_sem, 1)
            else:
                rs[st - 1].wait_recv()
                sbuf_ref[st] = (
                    part_ref[pl.ds(c * CHUNK, CHUNK), :]
                    + xg_ref[(st - 1) * CHUNK:st * CHUNK, :].astype(F32)
                ).astype(BF16)
            r = pltpu.make_async_remote_copy(
                src_ref=sbuf_ref.at[st],
                dst_ref=xg_ref.at[st * CHUNK:(st + 1) * CHUNK, :],
                send_sem=rs_send.at[st], recv_sem=rs_recv.at[st],
                device_id=(right,), device_id_type=pl.DeviceIdType.MESH,
            )
            r.start()
            rs.append(r)
        attn_chunk(my)
        rs[N_DEV - 2].wait_recv()
        out_ref[...] = (
            part_ref[pl.ds(my * CHUNK, CHUNK), :]
            + xg_ref[(N_DEV - 2) * CHUNK:(N_DEV - 1) * CHUNK, :].astype(F32))
        for r in rs:
            r.wait_send()

    out = pl.pallas_call(
        body,
        out_shape=jax.ShapeDtypeStruct((CHUNK, D), jnp.float32),
        in_specs=[pl.BlockSpec(memory_space=pltpu.VMEM)] * 5,
        out_specs=pl.BlockSpec(memory_space=pltpu.VMEM),
        scratch_shapes=[
            pltpu.VMEM((SEQ, D), BF16),
            pltpu.VMEM((SEQ, D), BF16),
            pltpu.VMEM((SEQ, D), BF16),
            pltpu.VMEM((SEQ, D), F32),
            pltpu.VMEM((CHUNK, D), F32),
            pltpu.VMEM((SEQ, D), F32),
            pltpu.VMEM((N_DEV - 1, CHUNK, D), BF16),
            pltpu.SemaphoreType.DMA((4,)),
            pltpu.SemaphoreType.DMA((4,)),
            pltpu.SemaphoreType.DMA((3,)),
            pltpu.SemaphoreType.DMA((3,)),
            pltpu.SemaphoreType.DMA((N_DEV - 1,)),
            pltpu.SemaphoreType.DMA((N_DEV - 1,)),
            pltpu.SemaphoreType.REGULAR,
        ],
        compiler_params=pltpu.CompilerParams(
            collective_id=0, vmem_limit_bytes=100 * 1024 * 1024),
    )(x2, Wq, Wo, Wk, Wv)
    return out.reshape(1, CHUNK, D)


# device time: 99611 ns/iter; 1.7849x vs baseline; 1.0397x over previous
import jax
import jax.numpy as jnp
from jax import lax
from jax.experimental import pallas as pl
from jax.experimental.pallas import tpu as pltpu

N_DEV = 8
SEQ = 2048
CHUNK = 256
D = 1024
NH = 8
DH = 128
SCALE = 0.08838834764831843
QSCALE = SCALE * 1.4426950408889634

F32 = jnp.float32
BF16 = jnp.bfloat16


def kernel(x, Wq, Wo, Wk, Wv):
    x2 = x.reshape(CHUNK, D)

    def body(x_ref, wq_ref, wo_ref, wk_ref, wv_ref, out_ref,
             xg_ref, qf_ref, kf_ref, vf_ref, oblk_ref, part_ref, sbuf_ref,
             ag_send, ag_recv, al_send, al_recv,
             rs_send, rs_recv, ls_send, ls_recv, mid_sem):
        my = lax.axis_index("i")
        right = lax.rem(my + 1, N_DEV)
        left = lax.rem(my + N_DEV - 1, N_DEV)

        barrier = pltpu.get_barrier_semaphore()
        for nbr in (left, right):
            pl.semaphore_signal(barrier, inc=1, device_id=(nbr,),
                                device_id_type=pl.DeviceIdType.MESH)
        pl.semaphore_wait(barrier, 2)

        def qkv_chunk(c):
            row = c * CHUNK
            xb = xg_ref[pl.ds(row, CHUNK), :].astype(F32)
            qf_ref[pl.ds(row, CHUNK), :] = (jnp.dot(
                xb, wq_ref[...], preferred_element_type=F32)
                * QSCALE).astype(BF16)
            kf_ref[pl.ds(row, CHUNK), :] = jnp.dot(
                xb, wk_ref[...], preferred_element_type=F32).astype(BF16)
            vf_ref[pl.ds(row, CHUNK), :] = jnp.dot(
                xb, wv_ref[...], preferred_element_type=F32)

        HR, HL = 4, 3

        def hop(src_chunk, sems, h, dst):
            d = pltpu.make_async_remote_copy(
                src_ref=xg_ref.at[pl.ds(src_chunk * CHUNK, CHUNK), :],
                dst_ref=xg_ref.at[pl.ds(src_chunk * CHUNK, CHUNK), :],
                send_sem=sems[0].at[h], recv_sem=sems[1].at[h],
                device_id=(dst,), device_id_type=pl.DeviceIdType.MESH,
            )
            d.start()
            return d

        xg_ref[pl.ds(my * CHUNK, CHUNK), :] = x_ref[...].astype(BF16)
        agr = [hop(my, (ag_send, ag_recv), 0, right)]
        agl = [hop(my, (al_send, al_recv), 0, left)]
        qkv_chunk(my)
        for h in range(HR):
            agr[h].wait_recv()
            cr = lax.rem(my - 1 - h + N_DEV, N_DEV)
            if h < HR - 1:
                agr.append(hop(cr, (ag_send, ag_recv), h + 1, right))
            qkv_chunk(cr)
            if h < HL:
                agl[h].wait_recv()
                cl = lax.rem(my + 1 + h, N_DEV)
                if h < HL - 1:
                    agl.append(hop(cl, (al_send, al_recv), h + 1, left))
                qkv_chunk(cl)
        for d in agr + agl:
            d.wait_send()

        for nbr in (left, right):
            pl.semaphore_signal(mid_sem, inc=1, device_id=(nbr,),
                                device_id_type=pl.DeviceIdType.MESH)

        def attn_chunk(c):
            row = c * CHUNK

            def hb(hh, carry):
                hcol = hh * DH
                q = qf_ref[pl.ds(row, CHUNK), pl.ds(hcol, DH)]
                s = lax.dot_general(
                    q, kf_ref[:, pl.ds(hcol, DH)], (((1,), (1,)), ((), ())),
                    preferred_element_type=F32)
                p = jnp.exp2(s)
                l = jnp.sum(p, axis=1, keepdims=True)
                ob = jnp.dot(p, vf_ref[:, pl.ds(hcol, DH)],
                             preferred_element_type=F32)
                oblk_ref[:, pl.ds(hcol, DH)] = ob * (1.0 / l)
                return carry

            lax.fori_loop(0, NH, hb, 0, unroll=2)
            part_ref[pl.ds(row, CHUNK), :] = jnp.dot(
                oblk_ref[...], wo_ref[...], preferred_element_type=F32)

        def rs_hop(st, slot, dst, sems):
            r = pltpu.make_async_remote_copy(
                src_ref=sbuf_ref.at[slot],
                dst_ref=xg_ref.at[slot * CHUNK:(slot + 1) * CHUNK, :],
                send_sem=sems[0].at[st], recv_sem=sems[1].at[st],
                device_id=(dst,), device_id_type=pl.DeviceIdType.MESH,
            )
            r.start()
            return r

        def part(c):
            return part_ref[pl.ds(c * CHUNK, CHUNK), :]

        def recv(slot):
            return xg_ref[slot * CHUNK:(slot + 1) * CHUNK, :].astype(F32)

        rsr, rsl = [], []
        for st in range(4):
            cr = lax.rem(my + 4 - st + N_DEV, N_DEV)
            attn_chunk(cr)
            if st < 3:
                cl = lax.rem(my - 3 + st + N_DEV, N_DEV)
                attn_chunk(cl)
            if st == 0:
                pl.semaphore_wait(mid_sem, 2)
                sbuf_ref[0] = part(cr).astype(BF16)
                sbuf_ref[4] = part(cl).astype(BF16)
            else:
                rsr[st - 1].wait_recv()
                sbuf_ref[st] = (part(cr) + recv(st - 1)).astype(BF16)
                if st < 3:
                    rsl[st - 1].wait_recv()
                    sbuf_ref[4 + st] = (part(cl) + recv(4 + st - 1)).astype(BF16)
            rsr.append(rs_hop(st, st, right, (rs_send, rs_recv)))
            if st < 3:
                rsl.append(rs_hop(st, 4 + st, left, (ls_send, ls_recv)))
        attn_chunk(my)
        rsr[3].wait_recv()
        rsl[2].wait_recv()
        out_ref[...] = part(my) + recv(3) + recv(6)
        for r in rsr + rsl:
            r.wait_send()

    out = pl.pallas_call(
        body,
        out_shape=jax.ShapeDtypeStruct((CHUNK, D), jnp.float32),
        in_specs=[pl.BlockSpec(memory_space=pltpu.VMEM)] * 5,
        out_specs=pl.BlockSpec(memory_space=pltpu.VMEM),
        scratch_shapes=[
            pltpu.VMEM((SEQ, D), BF16),
            pltpu.VMEM((SEQ, D), BF16),
            pltpu.VMEM((SEQ, D), BF16),
            pltpu.VMEM((SEQ, D), F32),
            pltpu.VMEM((CHUNK, D), F32),
            pltpu.VMEM((SEQ, D), F32),
            pltpu.VMEM((N_DEV - 1, CHUNK, D), BF16),
            pltpu.SemaphoreType.DMA((4,)),
            pltpu.SemaphoreType.DMA((4,)),
            pltpu.SemaphoreType.DMA((3,)),
            pltpu.SemaphoreType.DMA((3,)),
            pltpu.SemaphoreType.DMA((4,)),
            pltpu.SemaphoreType.DMA((4,)),
            pltpu.SemaphoreType.DMA((3,)),
            pltpu.SemaphoreType.DMA((3,)),
            pltpu.SemaphoreType.REGULAR,
        ],
        compiler_params=pltpu.CompilerParams(
            collective_id=0, vmem_limit_bytes=100 * 1024 * 1024),
    )(x2, Wq, Wo, Wk, Wv)
    return out.reshape(1, CHUNK, D)
